# Initial kernel scaffold; baseline (speedup 1.0000x reference)
import jax
import jax.numpy as jnp
from jax.experimental import pallas as pl


def kernel(feats, edge_index, W1, b1, W2, b2, eps1, eps2):
    raise NotImplementedError("write your pallas kernel here")


# speedup vs baseline: 9.2511x; 9.2511x over previous
import functools

import jax
import jax.numpy as jnp
from jax import lax
from jax.experimental import pallas as pl
from jax.experimental.pallas import tpu as pltpu
from jax.experimental.pallas import tpu_sc as plsc

N = 10000
E = 320000
D = 128

NC = 2
NS = 16
NW = NC * NS
EPT = E // NW
CHUNK = 80
NCH = EPT // CHUNK
SLAB = 624
REM = N - NS * SLAB
ZR = 16


def _segsum_body(g_hbm, srcf_hbm, dstr_hbm, out_hbm, agg_sh, sidx, didx,
                 zbuf, rows0, rows1, gsem0, gsem1, ssem0, ssem1):
    rows = (rows0, rows1)
    gsems = (gsem0, gsem1)
    ssems = (ssem0, ssem1)
    c = lax.axis_index("c")
    s = lax.axis_index("s")
    wid = c * NS + s

    zero16 = jnp.zeros((16,), jnp.float32)

    def zfill(i, carry):
        for k in range(D // 16):
            zbuf[i, pl.ds(k * 16, 16)] = zero16
        return carry

    lax.fori_loop(0, ZR, zfill, 0)
    for q in range(SLAB // ZR):
        pltpu.sync_copy(zbuf, agg_sh.at[pl.ds(s * SLAB + q * ZR, ZR)])

    @pl.when(s == NS - 1)
    def _zero_rem():
        pltpu.sync_copy(zbuf, agg_sh.at[pl.ds(NS * SLAB, REM)])

    pltpu.sync_copy(srcf_hbm.at[pl.ds(wid * EPT, EPT)], sidx)
    pltpu.sync_copy(dstr_hbm.at[wid], didx)
    plsc.subcore_barrier()

    def fire_gather(j, b):
        pltpu.async_copy(g_hbm.at[sidx.at[pl.ds(j * CHUNK, CHUNK)]],
                         rows[b], gsems[b])

    def wait_gather(j, b):
        pltpu.make_async_copy(g_hbm.at[sidx.at[pl.ds(j * CHUNK, CHUNK)]],
                              rows[b], gsems[b]).wait()

    def fire_scatter(j, b):
        pltpu.async_copy(rows[b], agg_sh.at[didx.at[j]], ssems[b], add=True)

    def wait_scatter(j, b):
        pltpu.make_async_copy(rows[b], agg_sh.at[didx.at[j]],
                              ssems[b]).wait()

    fire_gather(0, 0)
    fire_gather(1, 1)

    def group(g, carry):
        j0 = 2 * g
        for b in range(2):
            wait_gather(j0 + b, b)
            fire_scatter(j0 + b, b)
        for b in range(2):
            wait_scatter(j0 + b, b)
            fire_gather(j0 + 2 + b, b)
        return carry

    lax.fori_loop(0, (NCH - 3) // 2, group, 0)
    wait_gather(NCH - 3, 0)
    fire_scatter(NCH - 3, 0)
    wait_gather(NCH - 2, 1)
    fire_scatter(NCH - 2, 1)
    wait_scatter(NCH - 3, 0)
    fire_gather(NCH - 1, 0)
    wait_gather(NCH - 1, 0)
    fire_scatter(NCH - 1, 0)
    wait_scatter(NCH - 2, 1)
    wait_scatter(NCH - 1, 0)
    plsc.subcore_barrier()

    pltpu.sync_copy(agg_sh.at[pl.ds(s * SLAB, SLAB)],
                    out_hbm.at[c, pl.ds(s * SLAB, SLAB)])

    @pl.when(s == NS - 1)
    def _drain_rem():
        pltpu.sync_copy(agg_sh.at[pl.ds(NS * SLAB, REM)],
                        out_hbm.at[c, pl.ds(NS * SLAB, REM)])


def _make_segsum():
    mesh = plsc.VectorSubcoreMesh(core_axis_name="c", subcore_axis_name="s")
    scratch = [
        pltpu.VMEM_SHARED((N, D), jnp.float32),
        pltpu.VMEM((EPT,), jnp.int32),
        pltpu.VMEM((NCH, CHUNK), jnp.int32),
        pltpu.VMEM((ZR, D), jnp.float32),
        pltpu.VMEM((CHUNK, D), jnp.float32),
        pltpu.VMEM((CHUNK, D), jnp.float32),
        pltpu.SemaphoreType.DMA,
        pltpu.SemaphoreType.DMA,
        pltpu.SemaphoreType.DMA,
        pltpu.SemaphoreType.DMA,
    ]
    return pl.kernel(
        _segsum_body,
        out_type=jax.ShapeDtypeStruct((NC, N, D), jnp.float32),
        mesh=mesh,
        scratch_types=scratch,
    )


def _mm_body(x_ref, w_ref, o_ref):
    o_ref[...] = lax.dot_general(
        x_ref[...], w_ref[...], (((1,), (1,)), ((), ())),
        preferred_element_type=jnp.float32)


def _mm(x, w):
    return pl.pallas_call(
        _mm_body,
        grid=(10,),
        in_specs=[
            pl.BlockSpec((N // 10, D), lambda i: (i, 0)),
            pl.BlockSpec((D, D), lambda i: (0, 0)),
        ],
        out_specs=pl.BlockSpec((N // 10, D), lambda i: (i, 0)),
        out_shape=jax.ShapeDtypeStruct((N, D), jnp.float32),
    )(x, w)


def _combine_mm_body(scale_ref, g_ref, agg_ref, b_ref, w_ref, o_ref):
    z = (scale_ref[0] * g_ref[...] + agg_ref[0] + agg_ref[1]
         + b_ref[...][None, :])
    o_ref[...] = lax.dot_general(
        z, w_ref[...], (((1,), (1,)), ((), ())),
        preferred_element_type=jnp.float32)


def _combine_mm(scale, g, agg, b, w):
    return pl.pallas_call(
        _combine_mm_body,
        grid=(10,),
        in_specs=[
            pl.BlockSpec(memory_space=pltpu.SMEM),
            pl.BlockSpec((N // 10, D), lambda i: (i, 0)),
            pl.BlockSpec((NC, N // 10, D), lambda i: (0, i, 0)),
            pl.BlockSpec((D,), lambda i: (0,)),
            pl.BlockSpec((D, D), lambda i: (0, 0)),
        ],
        out_specs=pl.BlockSpec((N // 10, D), lambda i: (i, 0)),
        out_shape=jax.ShapeDtypeStruct((N, D), jnp.float32),
    )(scale, g, agg, b, w)


def _combine_body(scale_ref, g_ref, agg_ref, b_ref, o_ref):
    o_ref[...] = (scale_ref[0] * g_ref[...] + agg_ref[0] + agg_ref[1]
                  + b_ref[...][None, :])


def _combine(scale, g, agg, b):
    return pl.pallas_call(
        _combine_body,
        grid=(10,),
        in_specs=[
            pl.BlockSpec(memory_space=pltpu.SMEM),
            pl.BlockSpec((N // 10, D), lambda i: (i, 0)),
            pl.BlockSpec((NC, N // 10, D), lambda i: (0, i, 0)),
            pl.BlockSpec((D,), lambda i: (0,)),
        ],
        out_specs=pl.BlockSpec((N // 10, D), lambda i: (i, 0)),
        out_shape=jax.ShapeDtypeStruct((N, D), jnp.float32),
    )(scale, g, agg, b)


_segsum = _make_segsum()


def kernel(feats, edge_index, W1, b1, W2, b2, eps1, eps2):
    srcf = edge_index[0]
    dstr = edge_index[1].reshape(NW, NCH, CHUNK)
    scale1 = (1.0 + eps1).reshape(1)
    scale2 = (1.0 + eps2).reshape(1)
    g1 = _mm(feats, W1)
    agg1 = _segsum(g1, srcf, dstr)
    g2 = _combine_mm(scale1, g1, agg1, b1, W2)
    agg2 = _segsum(g2, srcf, dstr)
    return _combine(scale2, g2, agg2, b2)
